# Initial kernel scaffold; baseline (speedup 1.0000x reference)
import functools

import jax
import jax.numpy as jnp
from jax import lax
from jax.experimental import pallas as pl
from jax.experimental.pallas import tpu as pltpu
from jax.experimental.pallas import tpu_sc as plsc

N = 10000
E = 320000
D_IN = 128
HID = 128
C = 16
NEG_SLOPE = 0.2

NC = 2
NS = 16
L = 16
NW = NC * NS
EPT = E // NW
GP = EPT // L
RPS = N // NS
ZR = 125

RB = 1000
GRID = N // RB


def _dense1(x, W0, b0, W1, al1, ar1):
    def body(x_ref, w0_ref, b0_ref, w1_ref, al_ref, ar_ref,
             f_ref, el_ref, er_ref):
        h = jnp.dot(x_ref[...], w0_ref[...],
                    preferred_element_type=jnp.float32) + b0_ref[...][None, :]
        f = jnp.dot(h, w1_ref[...], preferred_element_type=jnp.float32)
        f_ref[...] = f
        el_ref[...] = jnp.sum(f * al_ref[...], axis=1, keepdims=True)
        er_ref[...] = jnp.sum(f * ar_ref[...], axis=1, keepdims=True)

    return pl.pallas_call(
        body,
        grid=(GRID,),
        in_specs=[
            pl.BlockSpec((RB, D_IN), lambda i: (i, 0)),
            pl.BlockSpec((D_IN, HID), lambda i: (0, 0)),
            pl.BlockSpec((HID,), lambda i: (0,)),
            pl.BlockSpec((HID, HID), lambda i: (0, 0)),
            pl.BlockSpec((1, HID), lambda i: (0, 0)),
            pl.BlockSpec((1, HID), lambda i: (0, 0)),
        ],
        out_specs=[
            pl.BlockSpec((RB, HID), lambda i: (i, 0)),
            pl.BlockSpec((RB, 1), lambda i: (i, 0)),
            pl.BlockSpec((RB, 1), lambda i: (i, 0)),
        ],
        out_shape=[
            jax.ShapeDtypeStruct((N, HID), jnp.float32),
            jax.ShapeDtypeStruct((N, 1), jnp.float32),
            jax.ShapeDtypeStruct((N, 1), jnp.float32),
        ],
    )(x, W0, b0, W1, al1, ar1)


def _dense2(accp, b1, W2, al2, ar2, resW2, b2):
    EXT = HID + L

    def body(acc_ref, b1_ref, w2_ref, al_ref, ar_ref, rw_ref, b2_ref,
             h1_ref, f2_ref, el_ref, er_ref, res_ref):
        a = acc_ref[0] + acc_ref[1]
        num = a[:, :HID]
        den = jnp.max(a[:, HID:EXT], axis=1, keepdims=True)
        o = num / jnp.maximum(den, 1e-9) + b1_ref[...][None, :]
        h1 = jnp.where(o > 0.0, o, jnp.exp(o) - 1.0)
        h1_ref[...] = h1
        f2 = jnp.dot(h1, w2_ref[...], preferred_element_type=jnp.float32)
        f2_ref[...] = f2
        el_ref[...] = jnp.sum(f2 * al_ref[...], axis=1, keepdims=True)
        er_ref[...] = jnp.sum(f2 * ar_ref[...], axis=1, keepdims=True)
        res_ref[...] = jnp.dot(h1, rw_ref[...],
                               preferred_element_type=jnp.float32) + b2_ref[...][None, :]

    return pl.pallas_call(
        body,
        grid=(GRID,),
        in_specs=[
            pl.BlockSpec((NC, RB, EXT), lambda i: (0, i, 0)),
            pl.BlockSpec((HID,), lambda i: (0,)),
            pl.BlockSpec((HID, C), lambda i: (0, 0)),
            pl.BlockSpec((1, C), lambda i: (0, 0)),
            pl.BlockSpec((1, C), lambda i: (0, 0)),
            pl.BlockSpec((HID, C), lambda i: (0, 0)),
            pl.BlockSpec((C,), lambda i: (0,)),
        ],
        out_specs=[
            pl.BlockSpec((RB, HID), lambda i: (i, 0)),
            pl.BlockSpec((RB, C), lambda i: (i, 0)),
            pl.BlockSpec((RB, 1), lambda i: (i, 0)),
            pl.BlockSpec((RB, 1), lambda i: (i, 0)),
            pl.BlockSpec((RB, C), lambda i: (i, 0)),
        ],
        out_shape=[
            jax.ShapeDtypeStruct((N, HID), jnp.float32),
            jax.ShapeDtypeStruct((N, C), jnp.float32),
            jax.ShapeDtypeStruct((N, 1), jnp.float32),
            jax.ShapeDtypeStruct((N, 1), jnp.float32),
            jax.ShapeDtypeStruct((N, C), jnp.float32),
        ],
    )(accp, b1, W2, al2, ar2, resW2, b2)


def _dense3(accp, res2):
    EXT = C + L

    def body(acc_ref, res_ref, out_ref):
        a = acc_ref[0] + acc_ref[1]
        num = a[:, :C]
        den = jnp.max(a[:, C:EXT], axis=1, keepdims=True)
        out_ref[...] = num / jnp.maximum(den, 1e-9) + res_ref[...]

    return pl.pallas_call(
        body,
        grid=(GRID,),
        in_specs=[
            pl.BlockSpec((NC, RB, EXT), lambda i: (0, i, 0)),
            pl.BlockSpec((RB, C), lambda i: (i, 0)),
        ],
        out_specs=pl.BlockSpec((RB, C), lambda i: (i, 0)),
        out_shape=jax.ShapeDtypeStruct((N, C), jnp.float32),
    )(accp, res2)


def _sc_edge(feat, el, er, src, dst, Dw):
    EXT = Dw + L
    mesh = plsc.VectorSubcoreMesh(core_axis_name="c", subcore_axis_name="s",
                                  num_cores=NC, num_subcores=NS)

    @functools.partial(
        pl.kernel,
        out_type=jax.ShapeDtypeStruct((NC, N, EXT), jnp.float32),
        mesh=mesh,
        scratch_types=[
            pltpu.VMEM((EPT,), jnp.int32),
            pltpu.VMEM((EPT,), jnp.int32),
            pltpu.VMEM((N,), jnp.float32),
            pltpu.VMEM((N,), jnp.float32),
            pltpu.VMEM((L, Dw), jnp.float32),
            pltpu.VMEM((L, EXT), jnp.float32),
            pltpu.VMEM((L,), jnp.float32),
            pltpu.VMEM((ZR, EXT), jnp.float32),
            pltpu.VMEM_SHARED((N, EXT), jnp.float32),
            pltpu.SemaphoreType.DMA,
        ],
    )
    def k(feat_hbm, el_hbm, er_hbm, src_hbm, dst_hbm, out_hbm,
          src_v, dst_v, el_v, er_v, rows_v, ext_v, ee_v, zb_v, acc_sh, sem):
        c = lax.axis_index("c")
        s = lax.axis_index("s")
        tid = c * NS + s
        ebase = tid * EPT
        pltpu.sync_copy(src_hbm.at[pl.ds(ebase, EPT)], src_v)
        pltpu.sync_copy(dst_hbm.at[pl.ds(ebase, EPT)], dst_v)
        pltpu.sync_copy(el_hbm, el_v)
        pltpu.sync_copy(er_hbm, er_v)

        zero = jnp.zeros((L,), jnp.float32)

        def zrow(i, carry):
            for j in range(EXT // L):
                zb_v[i, pl.ds(j * L, L)] = zero
            return carry

        lax.fori_loop(0, ZR, zrow, 0)
        for t in range(RPS // ZR):
            pltpu.sync_copy(zb_v, acc_sh.at[pl.ds(s * RPS + t * ZR, ZR)])
        plsc.subcore_barrier()

        def group(g, carry):
            base = pl.multiple_of(g * L, L)
            idxs = src_v[pl.ds(base, L)]
            idxd = dst_v[pl.ds(base, L)]
            elv = plsc.load_gather(el_v, [idxs])
            erv = plsc.load_gather(er_v, [idxd])
            e = elv + erv
            e = jnp.where(e >= 0.0, e, NEG_SLOPE * e)
            ee = jnp.exp(e)
            ee_v[...] = ee
            pltpu.async_copy(feat_hbm.at[idxs], rows_v, sem).wait()
            for i in range(L):
                sp = plsc.load_gather(ee_v, [jnp.full((L,), i, jnp.int32)])
                for j in range(Dw // L):
                    ext_v[i, pl.ds(j * L, L)] = rows_v[i, pl.ds(j * L, L)] * sp
                ext_v[i, pl.ds(Dw, L)] = sp
            pltpu.sync_copy(ext_v, acc_sh.at[idxd], add=True)
            return carry

        lax.fori_loop(0, GP, group, 0)
        plsc.subcore_barrier()
        pltpu.sync_copy(acc_sh.at[pl.ds(s * RPS, RPS)],
                        out_hbm.at[c, pl.ds(s * RPS, RPS)])

    return k(feat, el, er, src, dst)


def kernel(features_list, e_feat, edge_index, W0, b0, W1, al1, ar1, b1,
           W2, al2, ar2, b2, resW2):
    src = edge_index[0]
    dst = edge_index[1]
    feat1, el1, er1 = _dense1(features_list, W0, b0, W1, al1, ar1)
    acc1 = _sc_edge(feat1, el1.reshape(N), er1.reshape(N), src, dst, HID)
    h1, f2, el2, er2, res2 = _dense2(acc1, b1, W2, al2, ar2, resW2, b2)
    acc2 = _sc_edge(f2, el2.reshape(N), er2.reshape(N), src, dst, C)
    logits = _dense3(acc2, res2)
    return (logits, h1)


# speedup vs baseline: 18.5403x; 18.5403x over previous
import functools

import jax
import jax.numpy as jnp
from jax import lax
from jax.experimental import pallas as pl
from jax.experimental.pallas import tpu as pltpu
from jax.experimental.pallas import tpu_sc as plsc

N = 10000
E = 320000
D_IN = 128
HID = 128
C = 16
NEG_SLOPE = 0.2

NC = 2
NS = 16
L = 16
NW = NC * NS
EPT = E // NW
GP = EPT // L
N_PAD = 10240
RPS = N_PAD // NS
ZR = 16
EC = 2000

RB = 1000
GRID = N // RB


def _dense1(x, W0, b0, W1, al1, ar1):
    def body(x_ref, w0_ref, b0_ref, w1_ref, al_ref, ar_ref,
             f_ref, el_ref, er_ref):
        h = jnp.dot(x_ref[...], w0_ref[...],
                    preferred_element_type=jnp.float32) + b0_ref[...][None, :]
        f = jnp.dot(h, w1_ref[...], preferred_element_type=jnp.float32)
        f_ref[...] = f
        el_ref[...] = jnp.sum(f * al_ref[...], axis=1, keepdims=True)
        er_ref[...] = jnp.sum(f * ar_ref[...], axis=1, keepdims=True)

    return pl.pallas_call(
        body,
        grid=(GRID,),
        in_specs=[
            pl.BlockSpec((RB, D_IN), lambda i: (i, 0)),
            pl.BlockSpec((D_IN, HID), lambda i: (0, 0)),
            pl.BlockSpec((HID,), lambda i: (0,)),
            pl.BlockSpec((HID, HID), lambda i: (0, 0)),
            pl.BlockSpec((1, HID), lambda i: (0, 0)),
            pl.BlockSpec((1, HID), lambda i: (0, 0)),
        ],
        out_specs=[
            pl.BlockSpec((RB, HID), lambda i: (i, 0)),
            pl.BlockSpec((RB, 1), lambda i: (i, 0)),
            pl.BlockSpec((RB, 1), lambda i: (i, 0)),
        ],
        out_shape=[
            jax.ShapeDtypeStruct((N, HID), jnp.float32),
            jax.ShapeDtypeStruct((N, 1), jnp.float32),
            jax.ShapeDtypeStruct((N, 1), jnp.float32),
        ],
    )(x, W0, b0, W1, al1, ar1)


def _dense2(accp, b1, W2, al2, ar2, resW2, b2):
    EXT = HID + L

    def body(acc_ref, b1_ref, w2_ref, al_ref, ar_ref, rw_ref, b2_ref,
             h1_ref, f2_ref, el_ref, er_ref, res_ref):
        a = acc_ref[0] + acc_ref[1]
        num = a[:, :HID]
        den = jnp.max(a[:, HID:EXT], axis=1, keepdims=True)
        o = num / jnp.maximum(den, 1e-9) + b1_ref[...][None, :]
        h1 = jnp.where(o > 0.0, o, jnp.exp(o) - 1.0)
        h1_ref[...] = h1
        f2 = jnp.dot(h1, w2_ref[...], preferred_element_type=jnp.float32)
        f2_ref[...] = f2
        el_ref[...] = jnp.sum(f2 * al_ref[...], axis=1, keepdims=True)
        er_ref[...] = jnp.sum(f2 * ar_ref[...], axis=1, keepdims=True)
        res_ref[...] = jnp.dot(h1, rw_ref[...],
                               preferred_element_type=jnp.float32) + b2_ref[...][None, :]

    return pl.pallas_call(
        body,
        grid=(GRID,),
        in_specs=[
            pl.BlockSpec((NC, RB, EXT), lambda i: (0, i, 0)),
            pl.BlockSpec((HID,), lambda i: (0,)),
            pl.BlockSpec((HID, C), lambda i: (0, 0)),
            pl.BlockSpec((1, C), lambda i: (0, 0)),
            pl.BlockSpec((1, C), lambda i: (0, 0)),
            pl.BlockSpec((HID, C), lambda i: (0, 0)),
            pl.BlockSpec((C,), lambda i: (0,)),
        ],
        out_specs=[
            pl.BlockSpec((RB, HID), lambda i: (i, 0)),
            pl.BlockSpec((RB, C), lambda i: (i, 0)),
            pl.BlockSpec((RB, 1), lambda i: (i, 0)),
            pl.BlockSpec((RB, 1), lambda i: (i, 0)),
            pl.BlockSpec((RB, C), lambda i: (i, 0)),
        ],
        out_shape=[
            jax.ShapeDtypeStruct((N, HID), jnp.float32),
            jax.ShapeDtypeStruct((N, C), jnp.float32),
            jax.ShapeDtypeStruct((N, 1), jnp.float32),
            jax.ShapeDtypeStruct((N, 1), jnp.float32),
            jax.ShapeDtypeStruct((N, C), jnp.float32),
        ],
    )(accp, b1, W2, al2, ar2, resW2, b2)


def _dense3(accp, res2):
    EXT = C + L

    def body(acc_ref, res_ref, out_ref):
        a = acc_ref[0] + acc_ref[1]
        num = a[:, :C]
        den = jnp.max(a[:, C:EXT], axis=1, keepdims=True)
        out_ref[...] = num / jnp.maximum(den, 1e-9) + res_ref[...]

    return pl.pallas_call(
        body,
        grid=(GRID,),
        in_specs=[
            pl.BlockSpec((NC, RB, EXT), lambda i: (0, i, 0)),
            pl.BlockSpec((RB, C), lambda i: (i, 0)),
        ],
        out_specs=pl.BlockSpec((RB, C), lambda i: (i, 0)),
        out_shape=jax.ShapeDtypeStruct((N, C), jnp.float32),
    )(accp, res2)


def _sc_edge(feat, el, er, src, dst, Dw):
    EXT = Dw + L
    mesh = plsc.VectorSubcoreMesh(core_axis_name="c", subcore_axis_name="s",
                                  num_cores=NC, num_subcores=NS)

    @functools.partial(
        pl.kernel,
        out_type=jax.ShapeDtypeStruct((NC, N_PAD, EXT), jnp.float32),
        mesh=mesh,
        compiler_params=pltpu.CompilerParams(use_tc_tiling_on_sc=False,
                                             needs_layout_passes=False),
        scratch_types=[
            pltpu.VMEM((EC,), jnp.int32),
            pltpu.VMEM((EC,), jnp.int32),
            pltpu.VMEM((N,), jnp.float32),
            pltpu.VMEM((N,), jnp.float32),
            pltpu.VMEM((L, Dw), jnp.float32),
            pltpu.VMEM((L, EXT), jnp.float32),
            pltpu.VMEM((ZR, EXT), jnp.float32),
            pltpu.VMEM_SHARED((N_PAD, EXT), jnp.float32),
            pltpu.SemaphoreType.DMA,
        ],
    )
    def k(feat_hbm, el_hbm, er_hbm, src_hbm, dst_hbm, out_hbm,
          src_v, dst_v, el_v, er_v, rows_v, ext_v, zb_v, acc_sh, sem):
        c = lax.axis_index("c")
        s = lax.axis_index("s")
        tid = c * NS + s
        ebase = tid * EPT
        pltpu.sync_copy(el_hbm, el_v)
        pltpu.sync_copy(er_hbm, er_v)

        zero = jnp.zeros((L,), jnp.float32)

        def zrow(i, carry):
            for j in range(EXT // L):
                zb_v[i, pl.ds(j * L, L)] = zero
            return carry

        lax.fori_loop(0, ZR, zrow, 0)
        row0 = pl.multiple_of(s * RPS, 8)
        for t in range(RPS // ZR):
            pltpu.sync_copy(zb_v, acc_sh.at[pl.ds(row0 + t * ZR, ZR)])
        plsc.subcore_barrier()

        def chunk(ci, carry):
            cbase = pl.multiple_of(ebase + ci * EC, 8)
            pltpu.sync_copy(src_hbm.at[pl.ds(cbase, EC)], src_v)
            pltpu.sync_copy(dst_hbm.at[pl.ds(cbase, EC)], dst_v)

            def group(g, carry2):
                base = pl.multiple_of(g * L, L)
                idxs = src_v[pl.ds(base, L)]
                idxd = dst_v[pl.ds(base, L)]
                elv = plsc.load_gather(el_v, [idxs])
                erv = plsc.load_gather(er_v, [idxd])
                e = elv + erv
                e = jnp.where(e >= 0.0, e, NEG_SLOPE * e)
                ee = jnp.exp(e)
                pltpu.async_copy(feat_hbm.at[idxs], rows_v, sem).wait()
                for i in range(L):
                    sps = ee[i]
                    for j in range(Dw // L):
                        ext_v[i, pl.ds(j * L, L)] = (
                            rows_v[i, pl.ds(j * L, L)] * sps)
                    ext_v[i, pl.ds(Dw, L)] = jnp.full((L,), 1.0,
                                                      jnp.float32) * sps
                pltpu.sync_copy(ext_v, acc_sh.at[idxd], add=True)
                return carry2

            lax.fori_loop(0, EC // L, group, 0)
            return carry

        lax.fori_loop(0, EPT // EC, chunk, 0)
        plsc.subcore_barrier()
        pltpu.sync_copy(acc_sh.at[pl.ds(row0, RPS)],
                        out_hbm.at[c, pl.ds(row0, RPS)])

    return k(feat, el, er, src, dst)


def kernel(features_list, e_feat, edge_index, W0, b0, W1, al1, ar1, b1,
           W2, al2, ar2, b2, resW2):
    src = edge_index[0]
    dst = edge_index[1]
    feat1, el1, er1 = _dense1(features_list, W0, b0, W1, al1, ar1)
    acc1 = _sc_edge(feat1, el1.reshape(N), er1.reshape(N), src, dst, HID)
    h1, f2, el2, er2, res2 = _dense2(acc1, b1, W2, al2, ar2, resW2, b2)
    acc2 = _sc_edge(f2, el2.reshape(N), er2.reshape(N), src, dst, C)
    logits = _dense3(acc2, res2)
    return (logits, h1)


# speedup vs baseline: 34.1794x; 1.8435x over previous
import functools

import jax
import jax.numpy as jnp
from jax import lax
from jax.experimental import pallas as pl
from jax.experimental.pallas import tpu as pltpu
from jax.experimental.pallas import tpu_sc as plsc

N = 10000
E = 320000
D_IN = 128
HID = 128
C = 16
NEG_SLOPE = 0.2

NC = 2
NS = 16
L = 16
NW = NC * NS
EPT = E // NW
GP = EPT // L
N_PAD = 10240
RPS = N_PAD // NS
ZR = 16
EC = 2000

RB = 1000
GRID = N // RB


def _dense1(x, W0, b0, W1, al1, ar1):
    def body(x_ref, w0_ref, b0_ref, w1_ref, al_ref, ar_ref,
             f_ref, el_ref, er_ref):
        h = jnp.dot(x_ref[...], w0_ref[...],
                    preferred_element_type=jnp.float32) + b0_ref[...][None, :]
        f = jnp.dot(h, w1_ref[...], preferred_element_type=jnp.float32)
        f_ref[...] = f
        el_ref[...] = jnp.sum(f * al_ref[...], axis=1, keepdims=True)
        er_ref[...] = jnp.sum(f * ar_ref[...], axis=1, keepdims=True)

    return pl.pallas_call(
        body,
        grid=(GRID,),
        in_specs=[
            pl.BlockSpec((RB, D_IN), lambda i: (i, 0)),
            pl.BlockSpec((D_IN, HID), lambda i: (0, 0)),
            pl.BlockSpec((HID,), lambda i: (0,)),
            pl.BlockSpec((HID, HID), lambda i: (0, 0)),
            pl.BlockSpec((1, HID), lambda i: (0, 0)),
            pl.BlockSpec((1, HID), lambda i: (0, 0)),
        ],
        out_specs=[
            pl.BlockSpec((RB, HID), lambda i: (i, 0)),
            pl.BlockSpec((RB, 1), lambda i: (i, 0)),
            pl.BlockSpec((RB, 1), lambda i: (i, 0)),
        ],
        out_shape=[
            jax.ShapeDtypeStruct((N, HID), jnp.float32),
            jax.ShapeDtypeStruct((N, 1), jnp.float32),
            jax.ShapeDtypeStruct((N, 1), jnp.float32),
        ],
    )(x, W0, b0, W1, al1, ar1)


def _dense2(accp, b1, W2, al2, ar2, resW2, b2):
    EXT = HID + L

    def body(acc_ref, b1_ref, w2_ref, al_ref, ar_ref, rw_ref, b2_ref,
             h1_ref, f2_ref, el_ref, er_ref, res_ref):
        a = acc_ref[0] + acc_ref[1]
        num = a[:, :HID]
        den = jnp.max(a[:, HID:EXT], axis=1, keepdims=True)
        o = num / jnp.maximum(den, 1e-9) + b1_ref[...][None, :]
        h1 = jnp.where(o > 0.0, o, jnp.exp(o) - 1.0)
        h1_ref[...] = h1
        f2 = jnp.dot(h1, w2_ref[...], preferred_element_type=jnp.float32)
        f2_ref[...] = f2
        el_ref[...] = jnp.sum(f2 * al_ref[...], axis=1, keepdims=True)
        er_ref[...] = jnp.sum(f2 * ar_ref[...], axis=1, keepdims=True)
        res_ref[...] = jnp.dot(h1, rw_ref[...],
                               preferred_element_type=jnp.float32) + b2_ref[...][None, :]

    return pl.pallas_call(
        body,
        grid=(GRID,),
        in_specs=[
            pl.BlockSpec((NC, RB, EXT), lambda i: (0, i, 0)),
            pl.BlockSpec((HID,), lambda i: (0,)),
            pl.BlockSpec((HID, C), lambda i: (0, 0)),
            pl.BlockSpec((1, C), lambda i: (0, 0)),
            pl.BlockSpec((1, C), lambda i: (0, 0)),
            pl.BlockSpec((HID, C), lambda i: (0, 0)),
            pl.BlockSpec((C,), lambda i: (0,)),
        ],
        out_specs=[
            pl.BlockSpec((RB, HID), lambda i: (i, 0)),
            pl.BlockSpec((RB, C), lambda i: (i, 0)),
            pl.BlockSpec((RB, 1), lambda i: (i, 0)),
            pl.BlockSpec((RB, 1), lambda i: (i, 0)),
            pl.BlockSpec((RB, C), lambda i: (i, 0)),
        ],
        out_shape=[
            jax.ShapeDtypeStruct((N, HID), jnp.float32),
            jax.ShapeDtypeStruct((N, C), jnp.float32),
            jax.ShapeDtypeStruct((N, 1), jnp.float32),
            jax.ShapeDtypeStruct((N, 1), jnp.float32),
            jax.ShapeDtypeStruct((N, C), jnp.float32),
        ],
    )(accp, b1, W2, al2, ar2, resW2, b2)


def _dense3(accp, res2):
    EXT = C + L

    def body(acc_ref, res_ref, out_ref):
        a = acc_ref[0] + acc_ref[1]
        num = a[:, :C]
        den = jnp.max(a[:, C:EXT], axis=1, keepdims=True)
        out_ref[...] = num / jnp.maximum(den, 1e-9) + res_ref[...]

    return pl.pallas_call(
        body,
        grid=(GRID,),
        in_specs=[
            pl.BlockSpec((NC, RB, EXT), lambda i: (0, i, 0)),
            pl.BlockSpec((RB, C), lambda i: (i, 0)),
        ],
        out_specs=pl.BlockSpec((RB, C), lambda i: (i, 0)),
        out_shape=jax.ShapeDtypeStruct((N, C), jnp.float32),
    )(accp, res2)


def _sc_edge(feat, el, er, src, dst, Dw):
    EXT = Dw + L
    mesh = plsc.VectorSubcoreMesh(core_axis_name="c", subcore_axis_name="s",
                                  num_cores=NC, num_subcores=NS)

    @functools.partial(
        pl.kernel,
        out_type=jax.ShapeDtypeStruct((NC, N_PAD, EXT), jnp.float32),
        mesh=mesh,
        compiler_params=pltpu.CompilerParams(use_tc_tiling_on_sc=False,
                                             needs_layout_passes=False),
        scratch_types=[
            pltpu.VMEM((EC,), jnp.int32),
            pltpu.VMEM((EC,), jnp.int32),
            pltpu.VMEM((N,), jnp.float32),
            pltpu.VMEM((N,), jnp.float32),
            pltpu.VMEM((2, L, Dw), jnp.float32),
            pltpu.VMEM((2, L, EXT), jnp.float32),
            pltpu.VMEM((ZR, EXT), jnp.float32),
            pltpu.VMEM_SHARED((N_PAD, EXT), jnp.float32),
            pltpu.SemaphoreType.DMA,
            pltpu.SemaphoreType.DMA,
            pltpu.SemaphoreType.DMA,
            pltpu.SemaphoreType.DMA,
        ],
    )
    def k(feat_hbm, el_hbm, er_hbm, src_hbm, dst_hbm, out_hbm,
          src_v, dst_v, el_v, er_v, rows_v, ext_v, zb_v, acc_sh,
          gsem0, gsem1, ssem0, ssem1):
        gsems = (gsem0, gsem1)
        ssems = (ssem0, ssem1)
        c = lax.axis_index("c")
        s = lax.axis_index("s")
        tid = c * NS + s
        ebase = tid * EPT
        pltpu.sync_copy(el_hbm, el_v)
        pltpu.sync_copy(er_hbm, er_v)

        zero = jnp.zeros((L,), jnp.float32)

        def zrow(i, carry):
            for j in range(EXT // L):
                zb_v[i, pl.ds(j * L, L)] = zero
            return carry

        lax.fori_loop(0, ZR, zrow, 0)
        row0 = pl.multiple_of(s * RPS, 8)
        for t in range(RPS // ZR):
            pltpu.sync_copy(zb_v, acc_sh.at[pl.ds(row0 + t * ZR, ZR)])
        plsc.subcore_barrier()

        NG = EC // L

        def issue_gather(gl, b):
            idxs = src_v[pl.ds(pl.multiple_of(gl * L, L), L)]
            pltpu.async_copy(feat_hbm.at[idxs], rows_v.at[b], gsems[b])

        def wait_gather(b):
            pltpu.make_async_copy(feat_hbm.at[pl.ds(0, L), :],
                                  rows_v.at[b], gsems[b]).wait()

        def compute_group(gl, b):
            base = pl.multiple_of(gl * L, L)
            idxs = src_v[pl.ds(base, L)]
            idxd = dst_v[pl.ds(base, L)]
            elv = plsc.load_gather(el_v, [idxs])
            erv = plsc.load_gather(er_v, [idxd])
            e = elv + erv
            e = jnp.where(e >= 0.0, e, NEG_SLOPE * e)
            ee = jnp.exp(e)
            for i in range(L):
                sps = ee[i]
                for j in range(Dw // L):
                    ext_v[b, i, pl.ds(j * L, L)] = (
                        rows_v[b, i, pl.ds(j * L, L)] * sps)
                ext_v[b, i, pl.ds(Dw, L)] = jnp.full((L,), 1.0,
                                                     jnp.float32) * sps
            return idxd

        def drain_scatter(b, idxd):
            pltpu.make_async_copy(ext_v.at[b], acc_sh.at[idxd],
                                  ssems[b]).wait()

        def chunk(ci, carry):
            cbase = pl.multiple_of(ebase + ci * EC, 8)
            pltpu.sync_copy(src_hbm.at[pl.ds(cbase, EC)], src_v)
            pltpu.sync_copy(dst_hbm.at[pl.ds(cbase, EC)], dst_v)
            issue_gather(0, 0)
            issue_gather(1, 1)

            def pair(gg, carry2):
                for b in range(2):
                    gl = gg * 2 + b
                    wait_gather(b)
                    idxd = compute_group_pre(gg, gl, b)
                    pltpu.async_copy(ext_v.at[b], acc_sh.at[idxd],
                                     ssems[b], add=True)

                    @pl.when(gl + 2 < NG)
                    def _():
                        issue_gather(gl + 2, b)
                return carry2

            def compute_group_pre(gg, gl, b):
                @pl.when(gg > 0)
                def _():
                    drain_scatter(b, dst_v[pl.ds(0, L)])
                return compute_group(gl, b)

            lax.fori_loop(0, NG // 2, pair, 0)
            wait_gather(0)
            drain_scatter(0, dst_v[pl.ds(0, L)])
            idxd = compute_group(NG - 1, 0)
            pltpu.sync_copy(ext_v.at[0], acc_sh.at[idxd], add=True)
            drain_scatter(1, dst_v[pl.ds(0, L)])
            return carry

        lax.fori_loop(0, EPT // EC, chunk, 0)
        plsc.subcore_barrier()
        pltpu.sync_copy(acc_sh.at[pl.ds(row0, RPS)],
                        out_hbm.at[c, pl.ds(row0, RPS)])

    return k(feat, el, er, src, dst)


def kernel(features_list, e_feat, edge_index, W0, b0, W1, al1, ar1, b1,
           W2, al2, ar2, b2, resW2):
    src = edge_index[0]
    dst = edge_index[1]
    feat1, el1, er1 = _dense1(features_list, W0, b0, W1, al1, ar1)
    acc1 = _sc_edge(feat1, el1.reshape(N), er1.reshape(N), src, dst, HID)
    h1, f2, el2, er2, res2 = _dense2(acc1, b1, W2, al2, ar2, resW2, b2)
    acc2 = _sc_edge(f2, el2.reshape(N), er2.reshape(N), src, dst, C)
    logits = _dense3(acc2, res2)
    return (logits, h1)


# speedup vs baseline: 42.7000x; 1.2493x over previous
import functools

import jax
import jax.numpy as jnp
from jax import lax
from jax.experimental import pallas as pl
from jax.experimental.pallas import tpu as pltpu
from jax.experimental.pallas import tpu_sc as plsc

N = 10000
E = 320000
D_IN = 128
HID = 128
C = 16
NEG_SLOPE = 0.2

NC = 2
NS = 16
L = 16
NW = NC * NS
EPT = E // NW
GP = EPT // L
N_PAD = 10240
RPS = N_PAD // NS
ZR = 16
EC = 2000

RB = 1000
GRID = N // RB


def _dense1(x, W0, b0, W1, al1, ar1):
    def body(x_ref, w0_ref, b0_ref, w1_ref, al_ref, ar_ref,
             f_ref, el_ref, er_ref):
        h = jnp.dot(x_ref[...], w0_ref[...],
                    preferred_element_type=jnp.float32) + b0_ref[...][None, :]
        f = jnp.dot(h, w1_ref[...], preferred_element_type=jnp.float32)
        f_ref[...] = f
        el_ref[...] = jnp.sum(f * al_ref[...], axis=1, keepdims=True)
        er_ref[...] = jnp.sum(f * ar_ref[...], axis=1, keepdims=True)

    return pl.pallas_call(
        body,
        grid=(GRID,),
        in_specs=[
            pl.BlockSpec((RB, D_IN), lambda i: (i, 0)),
            pl.BlockSpec((D_IN, HID), lambda i: (0, 0)),
            pl.BlockSpec((HID,), lambda i: (0,)),
            pl.BlockSpec((HID, HID), lambda i: (0, 0)),
            pl.BlockSpec((1, HID), lambda i: (0, 0)),
            pl.BlockSpec((1, HID), lambda i: (0, 0)),
        ],
        out_specs=[
            pl.BlockSpec((RB, HID), lambda i: (i, 0)),
            pl.BlockSpec((RB, 1), lambda i: (i, 0)),
            pl.BlockSpec((RB, 1), lambda i: (i, 0)),
        ],
        out_shape=[
            jax.ShapeDtypeStruct((N, HID), jnp.float32),
            jax.ShapeDtypeStruct((N, 1), jnp.float32),
            jax.ShapeDtypeStruct((N, 1), jnp.float32),
        ],
    )(x, W0, b0, W1, al1, ar1)


def _dense2(accp, b1, W2, al2, ar2, resW2, b2):
    EXT = HID + L

    def body(acc_ref, b1_ref, w2_ref, al_ref, ar_ref, rw_ref, b2_ref,
             h1_ref, f2_ref, el_ref, er_ref, res_ref):
        a = acc_ref[0] + acc_ref[1]
        num = a[:, :HID]
        den = jnp.max(a[:, HID:EXT], axis=1, keepdims=True)
        o = num / jnp.maximum(den, 1e-9) + b1_ref[...][None, :]
        h1 = jnp.where(o > 0.0, o, jnp.exp(o) - 1.0)
        h1_ref[...] = h1
        f2 = jnp.dot(h1, w2_ref[...], preferred_element_type=jnp.float32)
        f2_ref[...] = f2
        el_ref[...] = jnp.sum(f2 * al_ref[...], axis=1, keepdims=True)
        er_ref[...] = jnp.sum(f2 * ar_ref[...], axis=1, keepdims=True)
        res_ref[...] = jnp.dot(h1, rw_ref[...],
                               preferred_element_type=jnp.float32) + b2_ref[...][None, :]

    return pl.pallas_call(
        body,
        grid=(GRID,),
        in_specs=[
            pl.BlockSpec((NC, RB, EXT), lambda i: (0, i, 0)),
            pl.BlockSpec((HID,), lambda i: (0,)),
            pl.BlockSpec((HID, C), lambda i: (0, 0)),
            pl.BlockSpec((1, C), lambda i: (0, 0)),
            pl.BlockSpec((1, C), lambda i: (0, 0)),
            pl.BlockSpec((HID, C), lambda i: (0, 0)),
            pl.BlockSpec((C,), lambda i: (0,)),
        ],
        out_specs=[
            pl.BlockSpec((RB, HID), lambda i: (i, 0)),
            pl.BlockSpec((RB, C), lambda i: (i, 0)),
            pl.BlockSpec((RB, 1), lambda i: (i, 0)),
            pl.BlockSpec((RB, 1), lambda i: (i, 0)),
            pl.BlockSpec((RB, C), lambda i: (i, 0)),
        ],
        out_shape=[
            jax.ShapeDtypeStruct((N, HID), jnp.float32),
            jax.ShapeDtypeStruct((N, C), jnp.float32),
            jax.ShapeDtypeStruct((N, 1), jnp.float32),
            jax.ShapeDtypeStruct((N, 1), jnp.float32),
            jax.ShapeDtypeStruct((N, C), jnp.float32),
        ],
    )(accp, b1, W2, al2, ar2, resW2, b2)


def _dense3(accp, res2):
    EXT = C + L

    def body(acc_ref, res_ref, out_ref):
        a = acc_ref[0] + acc_ref[1]
        num = a[:, :C]
        den = jnp.max(a[:, C:EXT], axis=1, keepdims=True)
        out_ref[...] = num / jnp.maximum(den, 1e-9) + res_ref[...]

    return pl.pallas_call(
        body,
        grid=(GRID,),
        in_specs=[
            pl.BlockSpec((NC, RB, EXT), lambda i: (0, i, 0)),
            pl.BlockSpec((RB, C), lambda i: (i, 0)),
        ],
        out_specs=pl.BlockSpec((RB, C), lambda i: (i, 0)),
        out_shape=jax.ShapeDtypeStruct((N, C), jnp.float32),
    )(accp, res2)


def _sc_edge(feat, el, er, src2, dst2, Dw, GE):
    EXT = Dw + L
    KV = GE // L
    GPT = EPT // GE
    NCH = 5
    CR = GPT // NCH
    mesh = plsc.VectorSubcoreMesh(core_axis_name="c", subcore_axis_name="s",
                                  num_cores=NC, num_subcores=NS)

    @functools.partial(
        pl.kernel,
        out_type=jax.ShapeDtypeStruct((NC, N_PAD, EXT), jnp.float32),
        mesh=mesh,
        compiler_params=pltpu.CompilerParams(use_tc_tiling_on_sc=False,
                                             needs_layout_passes=False),
        scratch_types=[
            pltpu.VMEM((CR, GE), jnp.int32),
            pltpu.VMEM((CR, GE), jnp.int32),
            pltpu.VMEM((N,), jnp.float32),
            pltpu.VMEM((N,), jnp.float32),
            pltpu.VMEM((2, GE, Dw), jnp.float32),
            pltpu.VMEM((2, GE, EXT), jnp.float32),
            pltpu.VMEM((ZR, EXT), jnp.float32),
            pltpu.VMEM_SHARED((N_PAD, EXT), jnp.float32),
            pltpu.SemaphoreType.DMA,
            pltpu.SemaphoreType.DMA,
            pltpu.SemaphoreType.DMA,
            pltpu.SemaphoreType.DMA,
        ],
    )
    def k(feat_hbm, el_hbm, er_hbm, src_hbm, dst_hbm, out_hbm,
          src_v, dst_v, el_v, er_v, rows_v, ext_v, zb_v, acc_sh,
          gsem0, gsem1, ssem0, ssem1):
        gsems = (gsem0, gsem1)
        ssems = (ssem0, ssem1)
        c = lax.axis_index("c")
        s = lax.axis_index("s")
        tid = c * NS + s
        gbase = tid * GPT
        pltpu.sync_copy(el_hbm, el_v)
        pltpu.sync_copy(er_hbm, er_v)

        zero = jnp.zeros((L,), jnp.float32)

        def zrow(i, carry):
            for j in range(EXT // L):
                zb_v[i, pl.ds(j * L, L)] = zero
            return carry

        lax.fori_loop(0, ZR, zrow, 0)
        row0 = pl.multiple_of(s * RPS, 8)
        for t in range(RPS // ZR):
            pltpu.sync_copy(zb_v, acc_sh.at[pl.ds(row0 + t * ZR, ZR)])
        plsc.subcore_barrier()

        def issue_gather(g, b):
            pltpu.async_copy(feat_hbm.at[src_v.at[g]], rows_v.at[b],
                             gsems[b])

        def wait_gather(b):
            pltpu.make_async_copy(feat_hbm.at[pl.ds(0, GE), :],
                                  rows_v.at[b], gsems[b]).wait()

        def compute_group(g, b):
            ees = []
            for kv in range(KV):
                idxs = src_v[g, pl.ds(kv * L, L)]
                idxd = dst_v[g, pl.ds(kv * L, L)]
                e = plsc.load_gather(el_v, [idxs]) + plsc.load_gather(
                    er_v, [idxd])
                e = jnp.where(e >= 0.0, e, NEG_SLOPE * e)
                ees.append(jnp.exp(e))
            sps = [ees[i // L][i % L] for i in range(GE)]
            one = jnp.full((L,), 1.0, jnp.float32)
            for i in range(GE):
                for j in range(Dw // L):
                    ext_v[b, i, pl.ds(j * L, L)] = (
                        rows_v[b, i, pl.ds(j * L, L)] * sps[i])
                ext_v[b, i, pl.ds(Dw, L)] = one * sps[i]

        def drain_scatter(b):
            pltpu.make_async_copy(ext_v.at[b], acc_sh.at[dst_v.at[0]],
                                  ssems[b]).wait()

        def chunk(ci, carry):
            crow = gbase + ci * CR
            pltpu.sync_copy(src_hbm.at[pl.ds(crow, CR)], src_v)
            pltpu.sync_copy(dst_hbm.at[pl.ds(crow, CR)], dst_v)
            issue_gather(0, 0)
            issue_gather(1, 1)

            def pair(gg, carry2):
                for b in range(2):
                    g = gg * 2 + b
                    wait_gather(b)

                    @pl.when(gg > 0)
                    def _():
                        drain_scatter(b)

                    compute_group(g, b)
                    pltpu.async_copy(ext_v.at[b], acc_sh.at[dst_v.at[g]],
                                     ssems[b], add=True)

                    @pl.when(g + 2 < CR)
                    def _():
                        issue_gather(g + 2, b)
                return carry2

            lax.fori_loop(0, CR // 2, pair, 0)
            wait_gather(0)
            drain_scatter(0)
            compute_group(CR - 1, 0)
            pltpu.sync_copy(ext_v.at[0], acc_sh.at[dst_v.at[CR - 1]],
                            add=True)
            drain_scatter(1)
            return carry

        lax.fori_loop(0, NCH, chunk, 0)
        plsc.subcore_barrier()
        pltpu.sync_copy(acc_sh.at[pl.ds(row0, RPS)],
                        out_hbm.at[c, pl.ds(row0, RPS)])

    return k(feat, el, er, src2, dst2)


def kernel(features_list, e_feat, edge_index, W0, b0, W1, al1, ar1, b1,
           W2, al2, ar2, b2, resW2):
    src = edge_index[0]
    dst = edge_index[1]
    GE1, GE2 = 16, 80
    feat1, el1, er1 = _dense1(features_list, W0, b0, W1, al1, ar1)
    acc1 = _sc_edge(feat1, el1.reshape(N), er1.reshape(N),
                    src.reshape(E // GE1, GE1), dst.reshape(E // GE1, GE1),
                    HID, GE1)
    h1, f2, el2, er2, res2 = _dense2(acc1, b1, W2, al2, ar2, resW2, b2)
    acc2 = _sc_edge(f2, el2.reshape(N), er2.reshape(N),
                    src.reshape(E // GE2, GE2), dst.reshape(E // GE2, GE2),
                    C, GE2)
    logits = _dense3(acc2, res2)
    return (logits, h1)


# speedup vs baseline: 56.7309x; 1.3286x over previous
import functools

import jax
import jax.numpy as jnp
from jax import lax
from jax.experimental import pallas as pl
from jax.experimental.pallas import tpu as pltpu
from jax.experimental.pallas import tpu_sc as plsc

N = 10000
E = 320000
D_IN = 128
HID = 128
C = 16
NEG_SLOPE = 0.2

NC = 2
NS = 16
L = 16
NW = NC * NS
EPT = E // NW
GP = EPT // L
N_PAD = 10240
RPS = N_PAD // NS
ZR = 16
EC = 2000

RB = 1000
GRID = N // RB


def _dense1(x, W0, b0, W1, al1, ar1):
    def body(x_ref, w0_ref, b0_ref, w1_ref, al_ref, ar_ref,
             f_ref, el_ref, er_ref):
        h = jnp.dot(x_ref[...], w0_ref[...],
                    preferred_element_type=jnp.float32) + b0_ref[...][None, :]
        f = jnp.dot(h, w1_ref[...], preferred_element_type=jnp.float32)
        f_ref[...] = f
        el_ref[...] = jnp.sum(f * al_ref[...], axis=1, keepdims=True)
        er_ref[...] = jnp.sum(f * ar_ref[...], axis=1, keepdims=True)

    return pl.pallas_call(
        body,
        grid=(GRID,),
        in_specs=[
            pl.BlockSpec((RB, D_IN), lambda i: (i, 0)),
            pl.BlockSpec((D_IN, HID), lambda i: (0, 0)),
            pl.BlockSpec((HID,), lambda i: (0,)),
            pl.BlockSpec((HID, HID), lambda i: (0, 0)),
            pl.BlockSpec((1, HID), lambda i: (0, 0)),
            pl.BlockSpec((1, HID), lambda i: (0, 0)),
        ],
        out_specs=[
            pl.BlockSpec((RB, HID), lambda i: (i, 0)),
            pl.BlockSpec((RB, 1), lambda i: (i, 0)),
            pl.BlockSpec((RB, 1), lambda i: (i, 0)),
        ],
        out_shape=[
            jax.ShapeDtypeStruct((N, HID), jnp.float32),
            jax.ShapeDtypeStruct((N, 1), jnp.float32),
            jax.ShapeDtypeStruct((N, 1), jnp.float32),
        ],
    )(x, W0, b0, W1, al1, ar1)


def _dense2(accp, b1, W2, al2, ar2, resW2, b2):
    HH = HID // 2
    EXT = HH + L

    def body(acc_ref, b1_ref, w2_ref, al_ref, ar_ref, rw_ref, b2_ref,
             h1_ref, f2_ref, el_ref, er_ref, res_ref):
        a0 = acc_ref[0]
        a1 = acc_ref[1]
        num = jnp.concatenate([a0[:, :HH], a1[:, :HH]], axis=1)
        den = jnp.max(a0[:, HH:EXT], axis=1, keepdims=True)
        o = num / jnp.maximum(den, 1e-9) + b1_ref[...][None, :]
        h1 = jnp.where(o > 0.0, o, jnp.exp(o) - 1.0)
        h1_ref[...] = h1
        f2 = jnp.dot(h1, w2_ref[...], preferred_element_type=jnp.float32)
        f2_ref[...] = f2
        el_ref[...] = jnp.sum(f2 * al_ref[...], axis=1, keepdims=True)
        er_ref[...] = jnp.sum(f2 * ar_ref[...], axis=1, keepdims=True)
        res_ref[...] = jnp.dot(h1, rw_ref[...],
                               preferred_element_type=jnp.float32) + b2_ref[...][None, :]

    return pl.pallas_call(
        body,
        grid=(GRID,),
        in_specs=[
            pl.BlockSpec((NC, RB, EXT), lambda i: (0, i, 0)),
            pl.BlockSpec((HID,), lambda i: (0,)),
            pl.BlockSpec((HID, C), lambda i: (0, 0)),
            pl.BlockSpec((1, C), lambda i: (0, 0)),
            pl.BlockSpec((1, C), lambda i: (0, 0)),
            pl.BlockSpec((HID, C), lambda i: (0, 0)),
            pl.BlockSpec((C,), lambda i: (0,)),
        ],
        out_specs=[
            pl.BlockSpec((RB, HID), lambda i: (i, 0)),
            pl.BlockSpec((RB, C), lambda i: (i, 0)),
            pl.BlockSpec((RB, 1), lambda i: (i, 0)),
            pl.BlockSpec((RB, 1), lambda i: (i, 0)),
            pl.BlockSpec((RB, C), lambda i: (i, 0)),
        ],
        out_shape=[
            jax.ShapeDtypeStruct((N, HID), jnp.float32),
            jax.ShapeDtypeStruct((N, C), jnp.float32),
            jax.ShapeDtypeStruct((N, 1), jnp.float32),
            jax.ShapeDtypeStruct((N, 1), jnp.float32),
            jax.ShapeDtypeStruct((N, C), jnp.float32),
        ],
    )(accp, b1, W2, al2, ar2, resW2, b2)


def _dense3(accp, res2):
    EXT = C + L

    def body(acc_ref, res_ref, out_ref):
        a = acc_ref[0] + acc_ref[1]
        num = a[:, :C]
        den = jnp.max(a[:, C:EXT], axis=1, keepdims=True)
        out_ref[...] = num / jnp.maximum(den, 1e-9) + res_ref[...]

    return pl.pallas_call(
        body,
        grid=(GRID,),
        in_specs=[
            pl.BlockSpec((NC, RB, EXT), lambda i: (0, i, 0)),
            pl.BlockSpec((RB, C), lambda i: (i, 0)),
        ],
        out_specs=pl.BlockSpec((RB, C), lambda i: (i, 0)),
        out_shape=jax.ShapeDtypeStruct((N, C), jnp.float32),
    )(accp, res2)


def _sc_edge(feat, el, er, src2, dst2, Dw, GE):
    EXT = Dw + L
    KV = GE // L
    GPT = EPT // GE
    NCH = 5
    CR = GPT // NCH
    mesh = plsc.VectorSubcoreMesh(core_axis_name="c", subcore_axis_name="s",
                                  num_cores=NC, num_subcores=NS)

    @functools.partial(
        pl.kernel,
        out_type=jax.ShapeDtypeStruct((NC, N_PAD, EXT), jnp.float32),
        mesh=mesh,
        compiler_params=pltpu.CompilerParams(use_tc_tiling_on_sc=False,
                                             needs_layout_passes=False),
        scratch_types=[
            pltpu.VMEM((CR, GE), jnp.int32),
            pltpu.VMEM((CR, GE), jnp.int32),
            pltpu.VMEM((N,), jnp.float32),
            pltpu.VMEM((N,), jnp.float32),
            pltpu.VMEM((2, GE, Dw), jnp.float32),
            pltpu.VMEM((2, GE, EXT), jnp.float32),
            pltpu.VMEM((ZR, EXT), jnp.float32),
            pltpu.VMEM_SHARED((N_PAD, EXT), jnp.float32),
            pltpu.SemaphoreType.DMA,
            pltpu.SemaphoreType.DMA,
            pltpu.SemaphoreType.DMA,
            pltpu.SemaphoreType.DMA,
        ],
    )
    def k(feat_hbm, el_hbm, er_hbm, src_hbm, dst_hbm, out_hbm,
          src_v, dst_v, el_v, er_v, rows_v, ext_v, zb_v, acc_sh,
          gsem0, gsem1, ssem0, ssem1):
        gsems = (gsem0, gsem1)
        ssems = (ssem0, ssem1)
        c = lax.axis_index("c")
        s = lax.axis_index("s")
        tid = c * NS + s
        gbase = tid * GPT
        pltpu.sync_copy(el_hbm, el_v)
        pltpu.sync_copy(er_hbm, er_v)

        zero = jnp.zeros((L,), jnp.float32)

        def zrow(i, carry):
            for j in range(EXT // L):
                zb_v[i, pl.ds(j * L, L)] = zero
            return carry

        lax.fori_loop(0, ZR, zrow, 0)
        row0 = pl.multiple_of(s * RPS, 8)
        for t in range(RPS // ZR):
            pltpu.sync_copy(zb_v, acc_sh.at[pl.ds(row0 + t * ZR, ZR)])
        plsc.subcore_barrier()

        def issue_gather(g, b):
            pltpu.async_copy(feat_hbm.at[src_v.at[g]], rows_v.at[b],
                             gsems[b])

        def wait_gather(b):
            pltpu.make_async_copy(feat_hbm.at[pl.ds(0, GE), :],
                                  rows_v.at[b], gsems[b]).wait()

        def compute_group(g, b):
            ees = []
            for kv in range(KV):
                idxs = src_v[g, pl.ds(kv * L, L)]
                idxd = dst_v[g, pl.ds(kv * L, L)]
                e = plsc.load_gather(el_v, [idxs]) + plsc.load_gather(
                    er_v, [idxd])
                e = jnp.where(e >= 0.0, e, NEG_SLOPE * e)
                ees.append(jnp.exp(e))
            sps = [ees[i // L][i % L] for i in range(GE)]
            one = jnp.full((L,), 1.0, jnp.float32)
            for i in range(GE):
                for j in range(Dw // L):
                    ext_v[b, i, pl.ds(j * L, L)] = (
                        rows_v[b, i, pl.ds(j * L, L)] * sps[i])
                ext_v[b, i, pl.ds(Dw, L)] = one * sps[i]

        def drain_scatter(b):
            pltpu.make_async_copy(ext_v.at[b], acc_sh.at[dst_v.at[0]],
                                  ssems[b]).wait()

        def chunk(ci, carry):
            crow = gbase + ci * CR
            pltpu.sync_copy(src_hbm.at[pl.ds(crow, CR)], src_v)
            pltpu.sync_copy(dst_hbm.at[pl.ds(crow, CR)], dst_v)
            issue_gather(0, 0)
            issue_gather(1, 1)

            def pair(gg, carry2):
                for b in range(2):
                    g = gg * 2 + b
                    wait_gather(b)

                    @pl.when(gg > 0)
                    def _():
                        drain_scatter(b)

                    compute_group(g, b)
                    pltpu.async_copy(ext_v.at[b], acc_sh.at[dst_v.at[g]],
                                     ssems[b], add=True)

                    @pl.when(g + 2 < CR)
                    def _():
                        issue_gather(g + 2, b)
                return carry2

            lax.fori_loop(0, CR // 2, pair, 0)
            wait_gather(0)
            drain_scatter(0)
            compute_group(CR - 1, 0)
            pltpu.sync_copy(ext_v.at[0], acc_sh.at[dst_v.at[CR - 1]],
                            add=True)
            drain_scatter(1)
            return carry

        lax.fori_loop(0, NCH, chunk, 0)
        plsc.subcore_barrier()
        pltpu.sync_copy(acc_sh.at[pl.ds(row0, RPS)],
                        out_hbm.at[c, pl.ds(row0, RPS)])

    return k(feat, el, er, src2, dst2)


def _sc_edge_split(featp, el, er, src2, dst2):
    GE = 80
    Dh = 64
    EXT = Dh + L
    KV = GE // L
    GPT = E // GE // NS
    NCH = 5
    CR = GPT // NCH
    mesh = plsc.VectorSubcoreMesh(core_axis_name="c", subcore_axis_name="s",
                                  num_cores=NC, num_subcores=NS)

    @functools.partial(
        pl.kernel,
        out_type=jax.ShapeDtypeStruct((NC, N_PAD, EXT), jnp.float32),
        mesh=mesh,
        compiler_params=pltpu.CompilerParams(use_tc_tiling_on_sc=False,
                                             needs_layout_passes=False),
        scratch_types=[
            pltpu.VMEM((CR, GE), jnp.int32),
            pltpu.VMEM((CR, GE), jnp.int32),
            pltpu.VMEM((2, GE), jnp.int32),
            pltpu.VMEM((N,), jnp.float32),
            pltpu.VMEM((N,), jnp.float32),
            pltpu.VMEM((2, GE, Dh), jnp.float32),
            pltpu.VMEM((2, GE, EXT), jnp.float32),
            pltpu.VMEM((ZR, EXT), jnp.float32),
            pltpu.VMEM_SHARED((N_PAD, EXT), jnp.float32),
            pltpu.SemaphoreType.DMA,
            pltpu.SemaphoreType.DMA,
            pltpu.SemaphoreType.DMA,
            pltpu.SemaphoreType.DMA,
        ],
    )
    def k(featp_hbm, el_hbm, er_hbm, src_hbm, dst_hbm, out_hbm,
          src_v, dst_v, gi_v, el_v, er_v, rows_v, ext_v, zb_v, acc_sh,
          gsem0, gsem1, ssem0, ssem1):
        gsems = (gsem0, gsem1)
        ssems = (ssem0, ssem1)
        c = lax.axis_index("c")
        s = lax.axis_index("s")
        gbase = s * GPT
        pltpu.sync_copy(el_hbm, el_v)
        pltpu.sync_copy(er_hbm, er_v)

        zero = jnp.zeros((L,), jnp.float32)

        def zrow(i, carry):
            for j in range(EXT // L):
                zb_v[i, pl.ds(j * L, L)] = zero
            return carry

        lax.fori_loop(0, ZR, zrow, 0)
        row0 = pl.multiple_of(s * RPS, 8)
        for t in range(RPS // ZR):
            pltpu.sync_copy(zb_v, acc_sh.at[pl.ds(row0 + t * ZR, ZR)])
        plsc.subcore_barrier()

        def issue_gather(g, b):
            for kv in range(KV):
                idxs = src_v[g, pl.ds(kv * L, L)]
                gi_v[b, pl.ds(kv * L, L)] = idxs * 2 + c
            pltpu.async_copy(featp_hbm.at[gi_v.at[b]], rows_v.at[b],
                             gsems[b])

        def wait_gather(b):
            pltpu.make_async_copy(featp_hbm.at[pl.ds(0, GE), :],
                                  rows_v.at[b], gsems[b]).wait()

        def compute_group(g, b):
            ees = []
            for kv in range(KV):
                idxs = src_v[g, pl.ds(kv * L, L)]
                idxd = dst_v[g, pl.ds(kv * L, L)]
                e = plsc.load_gather(el_v, [idxs]) + plsc.load_gather(
                    er_v, [idxd])
                e = jnp.where(e >= 0.0, e, NEG_SLOPE * e)
                ees.append(jnp.exp(e))
            sps = [ees[i // L][i % L] for i in range(GE)]
            one = jnp.full((L,), 1.0, jnp.float32)
            for i in range(GE):
                for j in range(Dh // L):
                    ext_v[b, i, pl.ds(j * L, L)] = (
                        rows_v[b, i, pl.ds(j * L, L)] * sps[i])
                ext_v[b, i, pl.ds(Dh, L)] = one * sps[i]

        def drain_scatter(b):
            pltpu.make_async_copy(ext_v.at[b], acc_sh.at[dst_v.at[0]],
                                  ssems[b]).wait()

        def chunk(ci, carry):
            crow = gbase + ci * CR
            pltpu.sync_copy(src_hbm.at[pl.ds(crow, CR)], src_v)
            pltpu.sync_copy(dst_hbm.at[pl.ds(crow, CR)], dst_v)
            issue_gather(0, 0)
            issue_gather(1, 1)

            def pair(gg, carry2):
                for b in range(2):
                    g = gg * 2 + b
                    wait_gather(b)

                    @pl.when(gg > 0)
                    def _():
                        drain_scatter(b)

                    compute_group(g, b)
                    pltpu.async_copy(ext_v.at[b], acc_sh.at[dst_v.at[g]],
                                     ssems[b], add=True)

                    @pl.when(g + 2 < CR)
                    def _():
                        issue_gather(g + 2, b)
                return carry2

            lax.fori_loop(0, CR // 2, pair, 0)
            drain_scatter(0)
            drain_scatter(1)
            return carry

        lax.fori_loop(0, NCH, chunk, 0)
        plsc.subcore_barrier()
        pltpu.sync_copy(acc_sh.at[pl.ds(row0, RPS)],
                        out_hbm.at[c, pl.ds(row0, RPS)])

    return k(featp, el, er, src2, dst2)


def kernel(features_list, e_feat, edge_index, W0, b0, W1, al1, ar1, b1,
           W2, al2, ar2, b2, resW2):
    GE = 80
    src2 = edge_index[0].reshape(E // GE, GE)
    dst2 = edge_index[1].reshape(E // GE, GE)
    feat1, el1, er1 = _dense1(features_list, W0, b0, W1, al1, ar1)
    acc1 = _sc_edge_split(feat1.reshape(2 * N, HID // 2), el1.reshape(N),
                          er1.reshape(N), src2, dst2)
    h1, f2, el2, er2, res2 = _dense2(acc1, b1, W2, al2, ar2, resW2, b2)
    acc2 = _sc_edge(f2, el2.reshape(N), er2.reshape(N), src2, dst2, C, GE)
    logits = _dense3(acc2, res2)
    return (logits, h1)
